# baseline (speedup 1.0000x reference)
import functools

import jax
import jax.numpy as jnp
from jax import lax
from jax.experimental import pallas as pl
from jax.experimental.pallas import tpu as pltpu
from jax.experimental.pallas import tpu_sc as plsc

N = 10000
E = 320000
NC, NS = 2, 16
NW = NC * NS
PER_TILE = E // NW
B = 128
NB, REM = divmod(PER_TILE, B)
NBUF = 3
NP = 10240
ZROWS = NP // NS

_mesh = functools.partial(
    plsc.VectorSubcoreMesh, core_axis_name="c", subcore_axis_name="s")


def _sc_degree(dst):
    @functools.partial(
        pl.kernel,
        out_type=jax.ShapeDtypeStruct((NC * NP, 16), jnp.float32),
        mesh=_mesh(),
        compiler_params=pltpu.CompilerParams(use_tc_tiling_on_sc=False),
        scratch_types=[
            pltpu.VMEM((PER_TILE,), jnp.int32),
            pltpu.VMEM((B, 16), jnp.float32),
            pltpu.VMEM_SHARED((NP, 16), jnp.float32),
            pltpu.SemaphoreType.DMA,
        ],
    )
    def k(dst_hbm, ones_hbm, zeros_hbm, out_hbm, idx_v, ones_v, acc, sem):
        cid = lax.axis_index("c")
        sid = lax.axis_index("s")
        wid = cid * NS + sid
        ci = pltpu.async_copy(
            dst_hbm.at[pl.ds(wid * PER_TILE, PER_TILE)], idx_v, sem)
        pltpu.sync_copy(zeros_hbm, acc.at[pl.ds(sid * ZROWS, ZROWS)])
        pltpu.sync_copy(ones_hbm, ones_v)
        ci.wait()
        plsc.subcore_barrier()

        @pl.loop(0, NB)
        def _(i):
            pltpu.sync_copy(ones_v, acc.at[idx_v.at[pl.ds(i * B, B)]],
                            add=True)

        pltpu.sync_copy(ones_v.at[pl.ds(0, REM)],
                        acc.at[idx_v.at[pl.ds(NB * B, REM)]], add=True)
        plsc.subcore_barrier()
        pltpu.sync_copy(
            acc.at[pl.ds(sid * ZROWS, ZROWS)],
            out_hbm.at[pl.ds(cid * NP + sid * ZROWS, ZROWS)])

    ones = jnp.ones((B, 16), jnp.float32)
    zeros = jnp.zeros((ZROWS, 16), jnp.float32)
    return k(dst, ones, zeros).reshape(NC, NP, 16)


def _sc_scatter_rows(g, src, dst, D):
    bb = 64 if D == 128 else 128
    nb, rem = divmod(PER_TILE, bb)

    @functools.partial(
        pl.kernel,
        out_type=jax.ShapeDtypeStruct((NC * NP, D), jnp.float32),
        mesh=_mesh(),
        compiler_params=pltpu.CompilerParams(use_tc_tiling_on_sc=False),
        scratch_types=[
            pltpu.VMEM((PER_TILE,), jnp.int32),
            pltpu.VMEM((PER_TILE,), jnp.int32),
        ] + [pltpu.VMEM((bb, D), jnp.float32)] * NBUF
          + [pltpu.VMEM((rem, D), jnp.float32)]
          + [pltpu.VMEM_SHARED((NP, D), jnp.float32)]
          + [pltpu.SemaphoreType.DMA] * (NBUF + 1),
    )
    def k(g_hbm, src_hbm, dst_hbm, zeros_hbm, out_hbm,
          src_v, dst_v, r0, r1, r2, rr, acc, isem, m0, m1, m2):
        rows = [r0, r1, r2]
        sems = [m0, m1, m2]
        cid = lax.axis_index("c")
        sid = lax.axis_index("s")
        wid = cid * NS + sid
        base = wid * PER_TILE
        cs = pltpu.async_copy(src_hbm.at[pl.ds(base, PER_TILE)], src_v, isem)
        cd = pltpu.async_copy(dst_hbm.at[pl.ds(base, PER_TILE)], dst_v, isem)
        pltpu.sync_copy(zeros_hbm, acc.at[pl.ds(sid * ZROWS, ZROWS)])
        cs.wait()
        cd.wait()
        plsc.subcore_barrier()

        def _sidx(i):
            return src_v.at[pl.ds(i * bb, bb)]

        def _didx(i):
            return dst_v.at[pl.ds(i * bb, bb)]

        for b in range(NBUF):
            pltpu.async_copy(g_hbm.at[_sidx(b)], rows[b], sems[b])

        @pl.loop(0, nb - NBUF, step=NBUF)
        def _(i):
            for b in range(NBUF):
                pltpu.make_async_copy(g_hbm.at[_sidx(i + b)],
                                      rows[b], sems[b]).wait()
                pltpu.sync_copy(rows[b], acc.at[_didx(i + b)], add=True)
                pltpu.async_copy(g_hbm.at[_sidx(i + b + NBUF)],
                                 rows[b], sems[b])

        for b in range(NBUF):
            i = nb - NBUF + b
            pltpu.make_async_copy(g_hbm.at[_sidx(i)], rows[b], sems[b]).wait()
            pltpu.sync_copy(rows[b], acc.at[_didx(i)], add=True)

        sr = src_v.at[pl.ds(nb * bb, rem)]
        dr = dst_v.at[pl.ds(nb * bb, rem)]
        pltpu.async_copy(g_hbm.at[sr], rr, isem).wait()
        pltpu.sync_copy(rr, acc.at[dr], add=True)

        plsc.subcore_barrier()
        pltpu.sync_copy(
            acc.at[pl.ds(sid * ZROWS, ZROWS)],
            out_hbm.at[pl.ds(cid * NP + sid * ZROWS, ZROWS)])

    zeros = jnp.zeros((ZROWS, D), jnp.float32)
    return k(g, src, dst, zeros).reshape(NC, NP, D)


def _sc_gather_pairs(z, lsrc, ldst, D):
    @functools.partial(
        pl.kernel,
        out_type=jax.ShapeDtypeStruct((E, 2 * D), jnp.bfloat16),
        mesh=_mesh(),
        compiler_params=pltpu.CompilerParams(use_tc_tiling_on_sc=False),
        scratch_types=[
            pltpu.VMEM((PER_TILE,), jnp.int32),
            pltpu.VMEM((PER_TILE,), jnp.int32),
        ] + [pltpu.VMEM((B, D), jnp.bfloat16)] * (2 * NBUF)
          + [pltpu.VMEM((REM, D), jnp.bfloat16)] * 2
          + [pltpu.SemaphoreType.DMA] * (NBUF + 1),
    )
    def k(z_hbm, s_hbm, d_hbm, ab_hbm,
          si_v, di_v, a0, a1, a2, b0, b1, b2, ar, br,
          isem, m0, m1, m2):
        av = [a0, a1, a2]
        bv = [b0, b1, b2]
        sems = [m0, m1, m2]
        cid = lax.axis_index("c")
        sid = lax.axis_index("s")
        wid = cid * NS + sid
        base = wid * PER_TILE
        cs = pltpu.async_copy(s_hbm.at[pl.ds(base, PER_TILE)], si_v, isem)
        cd = pltpu.async_copy(d_hbm.at[pl.ds(base, PER_TILE)], di_v, isem)
        cs.wait()
        cd.wait()

        def _sidx(i):
            return si_v.at[pl.ds(i * B, B)]

        def _didx(i):
            return di_v.at[pl.ds(i * B, B)]

        def _issue(i, b):
            pltpu.async_copy(z_hbm.at[_sidx(i)], av[b], sems[b])
            pltpu.async_copy(z_hbm.at[_didx(i)], bv[b], sems[b])

        def _drain(i, b):
            pltpu.make_async_copy(z_hbm.at[_sidx(i)], av[b], sems[b]).wait()
            pltpu.make_async_copy(z_hbm.at[_didx(i)], bv[b], sems[b]).wait()

        for b in range(NBUF):
            _issue(b, b)

        def _writeout(i, xa, xb, n):
            off = base + i * B
            pltpu.sync_copy(xa, ab_hbm.at[pl.ds(off, n), pl.ds(0, D)])
            pltpu.sync_copy(xb, ab_hbm.at[pl.ds(off, n), pl.ds(D, D)])

        @pl.loop(0, NB - NBUF, step=NBUF)
        def _(i):
            for b in range(NBUF):
                _drain(i + b, b)
                _writeout(i + b, av[b], bv[b], B)
                _issue(i + b + NBUF, b)

        for b in range(NBUF):
            i = NB - NBUF + b
            _drain(i, b)
            _writeout(i, av[b], bv[b], B)

        sr = si_v.at[pl.ds(NB * B, REM)]
        dr = di_v.at[pl.ds(NB * B, REM)]
        ca = pltpu.async_copy(z_hbm.at[sr], ar, isem)
        cb = pltpu.async_copy(z_hbm.at[dr], br, isem)
        ca.wait()
        cb.wait()
        _writeout(NB, ar, br, REM)

    return k(z, lsrc, ldst)


_RE = 5120


def _dot_body(ab_ref, r_ref):
    ab = ab_ref[...].astype(jnp.float32)
    r = jnp.sum(ab[:, :64] * ab[:, 64:], axis=1)
    r_ref[...] = r.reshape(_RE // 128, 128)


def _tc_dot(ab, D):
    r2 = pl.pallas_call(
        _dot_body,
        grid=((E + _RE - 1) // _RE,),
        in_specs=[pl.BlockSpec((_RE, 2 * D), lambda i: (i, 0))],
        out_specs=pl.BlockSpec((_RE // 128, 128), lambda i: (i, 0)),
        out_shape=jax.ShapeDtypeStruct((E // 128, 128), jnp.float32),
        compiler_params=pltpu.CompilerParams(
            dimension_semantics=("parallel",)),
    )(ab)
    return r2.reshape(E)


_R = 2000


def _dis_block(deg_ref):
    deg = deg_ref[0, :, 0:1] + deg_ref[1, :, 0:1] + 1.0
    return lax.rsqrt(deg)


def _enc1_body(x_ref, w_ref, deg_ref, h0_ref, g1_ref):
    dis = _dis_block(deg_ref)
    h0 = jnp.dot(x_ref[...], w_ref[...], preferred_element_type=jnp.float32)
    h0_ref[...] = h0
    g1_ref[...] = h0 * dis


def _tc_encode1(x, W1, degp):
    return pl.pallas_call(
        _enc1_body,
        grid=(N // _R,),
        in_specs=[
            pl.BlockSpec((_R, 128), lambda i: (i, 0)),
            pl.BlockSpec((128, 128), lambda i: (0, 0)),
            pl.BlockSpec((NC, _R, 16), lambda i: (0, i, 0)),
        ],
        out_specs=[
            pl.BlockSpec((_R, 128), lambda i: (i, 0)),
            pl.BlockSpec((_R, 128), lambda i: (i, 0)),
        ],
        out_shape=[jax.ShapeDtypeStruct((N, 128), jnp.float32)] * 2,
    )(x, W1, degp)


def _enc23_body(s1_ref, h0_ref, deg_ref, b1_ref, eps_ref, w2_ref,
                h2_ref, g2_ref):
    dis = _dis_block(deg_ref)
    s = s1_ref[0] + s1_ref[1]
    h = jnp.maximum(dis * s + (dis * dis) * h0_ref[...] + b1_ref[...], 0.0)
    mean = jnp.sum(h, axis=0, keepdims=True) * (1.0 / N)
    var = jnp.sum(h * h, axis=0, keepdims=True) * (1.0 / N) - mean * mean
    std = jnp.sqrt(jnp.maximum(var, 0.0))
    hs = h + (mean + std * eps_ref[...])
    h2 = jnp.dot(hs, w2_ref[...], preferred_element_type=jnp.float32)
    h2_ref[...] = h2
    g2_ref[...] = h2 * dis


def _tc_encode23(s1p, h0, degp, b1, eps, W2):
    return pl.pallas_call(
        _enc23_body,
        grid=(1,),
        in_specs=[
            pl.BlockSpec((NC, N, 128), lambda i: (0, 0, 0)),
            pl.BlockSpec((N, 128), lambda i: (0, 0)),
            pl.BlockSpec((NC, N, 16), lambda i: (0, 0, 0)),
            pl.BlockSpec((1, 128), lambda i: (0, 0)),
            pl.BlockSpec((1, 128), lambda i: (0, 0)),
            pl.BlockSpec((128, 64), lambda i: (0, 0)),
        ],
        out_specs=[
            pl.BlockSpec((N, 64), lambda i: (0, 0)),
            pl.BlockSpec((N, 64), lambda i: (0, 0)),
        ],
        out_shape=[jax.ShapeDtypeStruct((N, 64), jnp.float32)] * 2,
    )(s1p, h0, degp, b1.reshape(1, 128), eps.reshape(1, 128), W2)


def _enc4_body(s2_ref, h2_ref, deg_ref, b2_ref, z_ref):
    dis = _dis_block(deg_ref)
    s = s2_ref[0] + s2_ref[1]
    z = dis * s + (dis * dis) * h2_ref[...] + b2_ref[...]
    z_ref[...] = z.astype(jnp.bfloat16)


def _tc_encode4(s2p, h2, degp, b2):
    return pl.pallas_call(
        _enc4_body,
        grid=(N // _R,),
        in_specs=[
            pl.BlockSpec((NC, _R, 64), lambda i: (0, i, 0)),
            pl.BlockSpec((_R, 64), lambda i: (i, 0)),
            pl.BlockSpec((NC, _R, 16), lambda i: (0, i, 0)),
            pl.BlockSpec((1, 64), lambda i: (0, 0)),
        ],
        out_specs=pl.BlockSpec((_R, 64), lambda i: (i, 0)),
        out_shape=jax.ShapeDtypeStruct((N, 64), jnp.bfloat16),
    )(s2p, h2, degp, b2.reshape(1, 64))


def kernel(x, edge_index, edge_label_index, W1, b1, W2, b2):
    src = jnp.asarray(edge_index[0], jnp.int32)
    dst = jnp.asarray(edge_index[1], jnp.int32)
    lsrc = jnp.asarray(edge_label_index[0], jnp.int32)
    ldst = jnp.asarray(edge_label_index[1], jnp.int32)
    eps = jax.random.normal(jax.random.key(42), (128,), dtype=jnp.float32)

    degp = _sc_degree(dst)
    h0, g1 = _tc_encode1(x, W1, degp)
    s1p = _sc_scatter_rows(g1, src, dst, 128)
    h2, g2 = _tc_encode23(s1p, h0, degp, b1, eps, W2)
    s2p = _sc_scatter_rows(g2, src, dst, 64)
    z = _tc_encode4(s2p, h2, degp, b2)
    ab = _sc_gather_pairs(z, lsrc, ldst, 64)
    return _tc_dot(ab, 64)


# speedup vs baseline: 1.5418x; 1.5418x over previous
import functools

import jax
import jax.numpy as jnp
from jax import lax
from jax.experimental import pallas as pl
from jax.experimental.pallas import tpu as pltpu
from jax.experimental.pallas import tpu_sc as plsc

N = 10000
E = 320000
NC, NS = 2, 16
NW = NC * NS
PER_TILE = E // NW
B = 128
NB, REM = divmod(PER_TILE, B)
NBUF = 3
NP = 10240
ZROWS = NP // NS

_mesh = functools.partial(
    plsc.VectorSubcoreMesh, core_axis_name="c", subcore_axis_name="s")


def _sc_degree(dst):
    @functools.partial(
        pl.kernel,
        out_type=jax.ShapeDtypeStruct((NC * NP, 16), jnp.float32),
        mesh=_mesh(),
        compiler_params=pltpu.CompilerParams(use_tc_tiling_on_sc=False),
        scratch_types=[
            pltpu.VMEM((PER_TILE,), jnp.int32),
            pltpu.VMEM((B, 16), jnp.float32),
            pltpu.VMEM_SHARED((NP, 16), jnp.float32),
            pltpu.SemaphoreType.DMA,
        ],
    )
    def k(dst_hbm, ones_hbm, zeros_hbm, out_hbm, idx_v, ones_v, acc, sem):
        cid = lax.axis_index("c")
        sid = lax.axis_index("s")
        wid = cid * NS + sid
        ci = pltpu.async_copy(
            dst_hbm.at[pl.ds(wid * PER_TILE, PER_TILE)], idx_v, sem)
        pltpu.sync_copy(zeros_hbm, acc.at[pl.ds(sid * ZROWS, ZROWS)])
        pltpu.sync_copy(ones_hbm, ones_v)
        ci.wait()
        plsc.subcore_barrier()

        @pl.loop(0, NB)
        def _(i):
            pltpu.sync_copy(ones_v, acc.at[idx_v.at[pl.ds(i * B, B)]],
                            add=True)

        pltpu.sync_copy(ones_v.at[pl.ds(0, REM)],
                        acc.at[idx_v.at[pl.ds(NB * B, REM)]], add=True)
        plsc.subcore_barrier()
        pltpu.sync_copy(
            acc.at[pl.ds(sid * ZROWS, ZROWS)],
            out_hbm.at[pl.ds(cid * NP + sid * ZROWS, ZROWS)])

    ones = jnp.ones((B, 16), jnp.float32)
    zeros = jnp.zeros((ZROWS, 16), jnp.float32)
    return k(dst, ones, zeros).reshape(NC, NP, 16)


def _sc_scatter_rows(g, src, dst, D):
    bb = 64 if D == 128 else 128
    nb, rem = divmod(PER_TILE, bb)

    @functools.partial(
        pl.kernel,
        out_type=jax.ShapeDtypeStruct((NC * NP, D), jnp.float32),
        mesh=_mesh(),
        compiler_params=pltpu.CompilerParams(use_tc_tiling_on_sc=False),
        scratch_types=[
            pltpu.VMEM((PER_TILE,), jnp.int32),
            pltpu.VMEM((PER_TILE,), jnp.int32),
        ] + [pltpu.VMEM((bb, D), jnp.float32)] * NBUF
          + [pltpu.VMEM((rem, D), jnp.float32)]
          + [pltpu.VMEM_SHARED((NP, D), jnp.float32)]
          + [pltpu.SemaphoreType.DMA] * (NBUF + 1),
    )
    def k(g_hbm, src_hbm, dst_hbm, zeros_hbm, out_hbm,
          src_v, dst_v, r0, r1, r2, rr, acc, isem, m0, m1, m2):
        rows = [r0, r1, r2]
        sems = [m0, m1, m2]
        cid = lax.axis_index("c")
        sid = lax.axis_index("s")
        wid = cid * NS + sid
        base = wid * PER_TILE
        cs = pltpu.async_copy(src_hbm.at[pl.ds(base, PER_TILE)], src_v, isem)
        cd = pltpu.async_copy(dst_hbm.at[pl.ds(base, PER_TILE)], dst_v, isem)
        pltpu.sync_copy(zeros_hbm, acc.at[pl.ds(sid * ZROWS, ZROWS)])
        cs.wait()
        cd.wait()
        plsc.subcore_barrier()

        def _sidx(i):
            return src_v.at[pl.ds(i * bb, bb)]

        def _didx(i):
            return dst_v.at[pl.ds(i * bb, bb)]

        for b in range(NBUF):
            pltpu.async_copy(g_hbm.at[_sidx(b)], rows[b], sems[b])

        @pl.loop(0, nb - NBUF, step=NBUF)
        def _(i):
            for b in range(NBUF):
                pltpu.make_async_copy(g_hbm.at[_sidx(i + b)],
                                      rows[b], sems[b]).wait()
                pltpu.sync_copy(rows[b], acc.at[_didx(i + b)], add=True)
                pltpu.async_copy(g_hbm.at[_sidx(i + b + NBUF)],
                                 rows[b], sems[b])

        for b in range(NBUF):
            i = nb - NBUF + b
            pltpu.make_async_copy(g_hbm.at[_sidx(i)], rows[b], sems[b]).wait()
            pltpu.sync_copy(rows[b], acc.at[_didx(i)], add=True)

        sr = src_v.at[pl.ds(nb * bb, rem)]
        dr = dst_v.at[pl.ds(nb * bb, rem)]
        pltpu.async_copy(g_hbm.at[sr], rr, isem).wait()
        pltpu.sync_copy(rr, acc.at[dr], add=True)

        plsc.subcore_barrier()
        pltpu.sync_copy(
            acc.at[pl.ds(sid * ZROWS, ZROWS)],
            out_hbm.at[pl.ds(cid * NP + sid * ZROWS, ZROWS)])

    zeros = jnp.zeros((ZROWS, D), jnp.float32)
    return k(g, src, dst, zeros).reshape(NC, NP, D)


def _sc_gather_pairs(z, lsrc, ldst, D):
    @functools.partial(
        pl.kernel,
        out_type=jax.ShapeDtypeStruct((E, 2 * D), jnp.float32),
        mesh=_mesh(),
        compiler_params=pltpu.CompilerParams(use_tc_tiling_on_sc=False),
        scratch_types=[
            pltpu.VMEM((PER_TILE,), jnp.int32),
            pltpu.VMEM((PER_TILE,), jnp.int32),
        ] + [pltpu.VMEM((B, D), jnp.float32)] * (2 * NBUF)
          + [pltpu.VMEM((REM, D), jnp.float32)] * 2
          + [pltpu.SemaphoreType.DMA] * (NBUF + 1),
    )
    def k(z_hbm, s_hbm, d_hbm, ab_hbm,
          si_v, di_v, a0, a1, a2, b0, b1, b2, ar, br,
          isem, m0, m1, m2):
        av = [a0, a1, a2]
        bv = [b0, b1, b2]
        sems = [m0, m1, m2]
        cid = lax.axis_index("c")
        sid = lax.axis_index("s")
        wid = cid * NS + sid
        base = wid * PER_TILE
        cs = pltpu.async_copy(s_hbm.at[pl.ds(base, PER_TILE)], si_v, isem)
        cd = pltpu.async_copy(d_hbm.at[pl.ds(base, PER_TILE)], di_v, isem)
        cs.wait()
        cd.wait()

        def _sidx(i):
            return si_v.at[pl.ds(i * B, B)]

        def _didx(i):
            return di_v.at[pl.ds(i * B, B)]

        def _issue(i, b):
            pltpu.async_copy(z_hbm.at[_sidx(i)], av[b], sems[b])
            pltpu.async_copy(z_hbm.at[_didx(i)], bv[b], sems[b])

        def _drain(i, b):
            pltpu.make_async_copy(z_hbm.at[_sidx(i)], av[b], sems[b]).wait()
            pltpu.make_async_copy(z_hbm.at[_didx(i)], bv[b], sems[b]).wait()

        for b in range(NBUF):
            _issue(b, b)

        def _writeout(i, xa, xb, n):
            off = base + i * B
            pltpu.sync_copy(xa, ab_hbm.at[pl.ds(off, n), pl.ds(0, D)])
            pltpu.sync_copy(xb, ab_hbm.at[pl.ds(off, n), pl.ds(D, D)])

        @pl.loop(0, NB - NBUF, step=NBUF)
        def _(i):
            for b in range(NBUF):
                _drain(i + b, b)
                _writeout(i + b, av[b], bv[b], B)
                _issue(i + b + NBUF, b)

        for b in range(NBUF):
            i = NB - NBUF + b
            _drain(i, b)
            _writeout(i, av[b], bv[b], B)

        sr = si_v.at[pl.ds(NB * B, REM)]
        dr = di_v.at[pl.ds(NB * B, REM)]
        ca = pltpu.async_copy(z_hbm.at[sr], ar, isem)
        cb = pltpu.async_copy(z_hbm.at[dr], br, isem)
        ca.wait()
        cb.wait()
        _writeout(NB, ar, br, REM)

    return k(z, lsrc, ldst)


_RE = 5120


def _dot_body(ab_ref, r_ref):
    ab = ab_ref[...]
    r = jnp.sum(ab[:, :64] * ab[:, 64:], axis=1)
    r_ref[...] = r.reshape(_RE // 128, 128)


def _tc_dot(ab, D):
    r2 = pl.pallas_call(
        _dot_body,
        grid=((E + _RE - 1) // _RE,),
        in_specs=[pl.BlockSpec((_RE, 2 * D), lambda i: (i, 0))],
        out_specs=pl.BlockSpec((_RE // 128, 128), lambda i: (i, 0)),
        out_shape=jax.ShapeDtypeStruct((E // 128, 128), jnp.float32),
        compiler_params=pltpu.CompilerParams(
            dimension_semantics=("parallel",)),
    )(ab)
    return r2.reshape(E)


_R = 2000


def _dis_block(deg_ref):
    deg = deg_ref[0, :, 0:1] + deg_ref[1, :, 0:1] + 1.0
    return lax.rsqrt(deg)


def _enc1_body(x_ref, w_ref, deg_ref, h0_ref, g1_ref):
    dis = _dis_block(deg_ref)
    h0 = jnp.dot(x_ref[...], w_ref[...], preferred_element_type=jnp.float32)
    h0_ref[...] = h0
    g1_ref[...] = h0 * dis


def _tc_encode1(x, W1, degp):
    return pl.pallas_call(
        _enc1_body,
        grid=(N // _R,),
        in_specs=[
            pl.BlockSpec((_R, 128), lambda i: (i, 0)),
            pl.BlockSpec((128, 128), lambda i: (0, 0)),
            pl.BlockSpec((NC, _R, 16), lambda i: (0, i, 0)),
        ],
        out_specs=[
            pl.BlockSpec((_R, 128), lambda i: (i, 0)),
            pl.BlockSpec((_R, 128), lambda i: (i, 0)),
        ],
        out_shape=[jax.ShapeDtypeStruct((N, 128), jnp.float32)] * 2,
    )(x, W1, degp)


def _enc23_body(s1_ref, h0_ref, deg_ref, b1_ref, eps_ref, w2_ref,
                h2_ref, g2_ref):
    dis = _dis_block(deg_ref)
    s = s1_ref[0] + s1_ref[1]
    h = jnp.maximum(dis * s + (dis * dis) * h0_ref[...] + b1_ref[...], 0.0)
    mean = jnp.sum(h, axis=0, keepdims=True) * (1.0 / N)
    var = jnp.sum(h * h, axis=0, keepdims=True) * (1.0 / N) - mean * mean
    std = jnp.sqrt(jnp.maximum(var, 0.0))
    hs = h + (mean + std * eps_ref[...])
    h2 = jnp.dot(hs, w2_ref[...], preferred_element_type=jnp.float32)
    h2_ref[...] = h2
    g2_ref[...] = h2 * dis


def _tc_encode23(s1p, h0, degp, b1, eps, W2):
    return pl.pallas_call(
        _enc23_body,
        grid=(1,),
        in_specs=[
            pl.BlockSpec((NC, N, 128), lambda i: (0, 0, 0)),
            pl.BlockSpec((N, 128), lambda i: (0, 0)),
            pl.BlockSpec((NC, N, 16), lambda i: (0, 0, 0)),
            pl.BlockSpec((1, 128), lambda i: (0, 0)),
            pl.BlockSpec((1, 128), lambda i: (0, 0)),
            pl.BlockSpec((128, 64), lambda i: (0, 0)),
        ],
        out_specs=[
            pl.BlockSpec((N, 64), lambda i: (0, 0)),
            pl.BlockSpec((N, 64), lambda i: (0, 0)),
        ],
        out_shape=[jax.ShapeDtypeStruct((N, 64), jnp.float32)] * 2,
    )(s1p, h0, degp, b1.reshape(1, 128), eps.reshape(1, 128), W2)


def _enc4_body(s2_ref, h2_ref, deg_ref, b2_ref, z_ref):
    dis = _dis_block(deg_ref)
    s = s2_ref[0] + s2_ref[1]
    z_ref[...] = dis * s + (dis * dis) * h2_ref[...] + b2_ref[...]


def _tc_encode4(s2p, h2, degp, b2):
    return pl.pallas_call(
        _enc4_body,
        grid=(N // _R,),
        in_specs=[
            pl.BlockSpec((NC, _R, 64), lambda i: (0, i, 0)),
            pl.BlockSpec((_R, 64), lambda i: (i, 0)),
            pl.BlockSpec((NC, _R, 16), lambda i: (0, i, 0)),
            pl.BlockSpec((1, 64), lambda i: (0, 0)),
        ],
        out_specs=pl.BlockSpec((_R, 64), lambda i: (i, 0)),
        out_shape=jax.ShapeDtypeStruct((N, 64), jnp.float32),
    )(s2p, h2, degp, b2.reshape(1, 64))


def kernel(x, edge_index, edge_label_index, W1, b1, W2, b2):
    src = jnp.asarray(edge_index[0], jnp.int32)
    dst = jnp.asarray(edge_index[1], jnp.int32)
    lsrc = jnp.asarray(edge_label_index[0], jnp.int32)
    ldst = jnp.asarray(edge_label_index[1], jnp.int32)
    eps = jax.random.normal(jax.random.key(42), (128,), dtype=jnp.float32)

    degp = _sc_degree(dst)
    h0, g1 = _tc_encode1(x, W1, degp)
    s1p = _sc_scatter_rows(g1, src, dst, 128)
    h2, g2 = _tc_encode23(s1p, h0, degp, b1, eps, W2)
    s2p = _sc_scatter_rows(g2, src, dst, 64)
    z = _tc_encode4(s2p, h2, degp, b2)
    ab = _sc_gather_pairs(z, lsrc, ldst, 64)
    return _tc_dot(ab, 64)


# speedup vs baseline: 1.5856x; 1.0284x over previous
import functools

import jax
import jax.numpy as jnp
from jax import lax
from jax.experimental import pallas as pl
from jax.experimental.pallas import tpu as pltpu
from jax.experimental.pallas import tpu_sc as plsc

N = 10000
E = 320000
NC, NS = 2, 16
NW = NC * NS
PER_TILE = E // NW
B = 128
NB, REM = divmod(PER_TILE, B)
NBUF = 3
NP = 10240
ZROWS = NP // NS

_mesh = functools.partial(
    plsc.VectorSubcoreMesh, core_axis_name="c", subcore_axis_name="s")


def _sc_degree(dst):
    @functools.partial(
        pl.kernel,
        out_type=jax.ShapeDtypeStruct((NC * NP, 16), jnp.float32),
        mesh=_mesh(),
        compiler_params=pltpu.CompilerParams(use_tc_tiling_on_sc=False),
        scratch_types=[
            pltpu.VMEM((PER_TILE,), jnp.int32),
            pltpu.VMEM((B, 16), jnp.float32),
            pltpu.VMEM_SHARED((NP, 16), jnp.float32),
            pltpu.SemaphoreType.DMA,
        ],
    )
    def k(dst_hbm, ones_hbm, zeros_hbm, out_hbm, idx_v, ones_v, acc, sem):
        cid = lax.axis_index("c")
        sid = lax.axis_index("s")
        wid = cid * NS + sid
        ci = pltpu.async_copy(
            dst_hbm.at[pl.ds(wid * PER_TILE, PER_TILE)], idx_v, sem)
        pltpu.sync_copy(zeros_hbm, acc.at[pl.ds(sid * ZROWS, ZROWS)])
        pltpu.sync_copy(ones_hbm, ones_v)
        ci.wait()
        plsc.subcore_barrier()

        @pl.loop(0, NB)
        def _(i):
            pltpu.sync_copy(ones_v, acc.at[idx_v.at[pl.ds(i * B, B)]],
                            add=True)

        pltpu.sync_copy(ones_v.at[pl.ds(0, REM)],
                        acc.at[idx_v.at[pl.ds(NB * B, REM)]], add=True)
        plsc.subcore_barrier()
        pltpu.sync_copy(
            acc.at[pl.ds(sid * ZROWS, ZROWS)],
            out_hbm.at[pl.ds(cid * NP + sid * ZROWS, ZROWS)])

    ones = jnp.ones((B, 16), jnp.float32)
    zeros = jnp.zeros((ZROWS, 16), jnp.float32)
    return k(dst, ones, zeros).reshape(NC, NP, 16)


def _sc_scatter_rows(g, src, dst, D):
    bb = 64 if D == 128 else 128
    nb, rem = divmod(PER_TILE, bb)

    @functools.partial(
        pl.kernel,
        out_type=jax.ShapeDtypeStruct((NC * NP, D), jnp.float32),
        mesh=_mesh(),
        compiler_params=pltpu.CompilerParams(use_tc_tiling_on_sc=False),
        scratch_types=[
            pltpu.VMEM((PER_TILE,), jnp.int32),
            pltpu.VMEM((PER_TILE,), jnp.int32),
        ] + [pltpu.VMEM((bb, D), jnp.float32)] * NBUF
          + [pltpu.VMEM((rem, D), jnp.float32)]
          + [pltpu.VMEM_SHARED((NP, D), jnp.float32)]
          + [pltpu.SemaphoreType.DMA] * (NBUF + 1),
    )
    def k(g_hbm, src_hbm, dst_hbm, zeros_hbm, out_hbm,
          src_v, dst_v, r0, r1, r2, rr, acc, isem, m0, m1, m2):
        rows = [r0, r1, r2]
        sems = [m0, m1, m2]
        cid = lax.axis_index("c")
        sid = lax.axis_index("s")
        wid = cid * NS + sid
        base = wid * PER_TILE
        cs = pltpu.async_copy(src_hbm.at[pl.ds(base, PER_TILE)], src_v, isem)
        cd = pltpu.async_copy(dst_hbm.at[pl.ds(base, PER_TILE)], dst_v, isem)
        pltpu.sync_copy(zeros_hbm, acc.at[pl.ds(sid * ZROWS, ZROWS)])
        cs.wait()
        cd.wait()
        plsc.subcore_barrier()

        def _sidx(i):
            return src_v.at[pl.ds(i * bb, bb)]

        def _didx(i):
            return dst_v.at[pl.ds(i * bb, bb)]

        for b in range(NBUF):
            pltpu.async_copy(g_hbm.at[_sidx(b)], rows[b], sems[b])

        @pl.loop(0, nb - NBUF, step=NBUF)
        def _(i):
            for b in range(NBUF):
                pltpu.make_async_copy(g_hbm.at[_sidx(i + b)],
                                      rows[b], sems[b]).wait()
                pltpu.sync_copy(rows[b], acc.at[_didx(i + b)], add=True)
                pltpu.async_copy(g_hbm.at[_sidx(i + b + NBUF)],
                                 rows[b], sems[b])

        for b in range(NBUF):
            i = nb - NBUF + b
            pltpu.make_async_copy(g_hbm.at[_sidx(i)], rows[b], sems[b]).wait()
            pltpu.sync_copy(rows[b], acc.at[_didx(i)], add=True)

        sr = src_v.at[pl.ds(nb * bb, rem)]
        dr = dst_v.at[pl.ds(nb * bb, rem)]
        pltpu.async_copy(g_hbm.at[sr], rr, isem).wait()
        pltpu.sync_copy(rr, acc.at[dr], add=True)

        plsc.subcore_barrier()
        pltpu.sync_copy(
            acc.at[pl.ds(sid * ZROWS, ZROWS)],
            out_hbm.at[pl.ds(cid * NP + sid * ZROWS, ZROWS)])

    zeros = jnp.zeros((ZROWS, D), jnp.float32)
    return k(g, src, dst, zeros).reshape(NC, NP, D)


def _sc_gather_pairs(z, lsrc, ldst, D, eo, ne):
    per_tile = ne // NW
    nb, rem = divmod(per_tile, B)
    nstep = NBUF * (nb // NBUF)

    @functools.partial(
        pl.kernel,
        out_type=jax.ShapeDtypeStruct((ne, 2 * D), jnp.float32),
        mesh=_mesh(),
        compiler_params=pltpu.CompilerParams(use_tc_tiling_on_sc=False),
        scratch_types=[
            pltpu.VMEM((per_tile,), jnp.int32),
            pltpu.VMEM((per_tile,), jnp.int32),
        ] + [pltpu.VMEM((B, D), jnp.float32)] * (2 * NBUF)
          + [pltpu.VMEM((rem, D), jnp.float32)] * 2
          + [pltpu.SemaphoreType.DMA] * (NBUF + 1),
    )
    def k(z_hbm, s_hbm, d_hbm, ab_hbm,
          si_v, di_v, a0, a1, a2, b0, b1, b2, ar, br,
          isem, m0, m1, m2):
        av = [a0, a1, a2]
        bv = [b0, b1, b2]
        sems = [m0, m1, m2]
        cid = lax.axis_index("c")
        sid = lax.axis_index("s")
        wid = cid * NS + sid
        base = wid * per_tile
        cs = pltpu.async_copy(s_hbm.at[pl.ds(eo + base, per_tile)], si_v, isem)
        cd = pltpu.async_copy(d_hbm.at[pl.ds(eo + base, per_tile)], di_v, isem)
        cs.wait()
        cd.wait()

        def _sidx(i):
            return si_v.at[pl.ds(i * B, B)]

        def _didx(i):
            return di_v.at[pl.ds(i * B, B)]

        def _issue(i, b):
            pltpu.async_copy(z_hbm.at[_sidx(i)], av[b], sems[b])
            pltpu.async_copy(z_hbm.at[_didx(i)], bv[b], sems[b])

        def _drain(i, b):
            pltpu.make_async_copy(z_hbm.at[_sidx(i)], av[b], sems[b]).wait()
            pltpu.make_async_copy(z_hbm.at[_didx(i)], bv[b], sems[b]).wait()

        for b in range(NBUF):
            _issue(b, b)

        def _writeout(i, xa, xb, n):
            off = base + i * B
            pltpu.sync_copy(xa, ab_hbm.at[pl.ds(off, n), pl.ds(0, D)])
            pltpu.sync_copy(xb, ab_hbm.at[pl.ds(off, n), pl.ds(D, D)])

        @pl.loop(0, nstep - NBUF, step=NBUF)
        def _(i):
            for b in range(NBUF):
                _drain(i + b, b)
                _writeout(i + b, av[b], bv[b], B)
                _issue(i + b + NBUF, b)

        for b in range(NBUF):
            i = nstep - NBUF + b
            _drain(i, b)
            _writeout(i, av[b], bv[b], B)

        sr = si_v.at[pl.ds(nb * B, rem)]
        dr = di_v.at[pl.ds(nb * B, rem)]
        ca = pltpu.async_copy(z_hbm.at[sr], ar, isem)
        cb = pltpu.async_copy(z_hbm.at[dr], br, isem)
        ca.wait()
        cb.wait()
        _writeout(nb, ar, br, rem)

    return k(z, lsrc, ldst)


_RE = 5120


def _dot_body(ab_ref, r_ref):
    ab = ab_ref[...]
    r = jnp.sum(ab[:, :64] * ab[:, 64:], axis=1)
    r_ref[...] = r.reshape(_RE // 128, 128)


def _tc_dot(ab, D, ne):
    r2 = pl.pallas_call(
        _dot_body,
        grid=((ne + _RE - 1) // _RE,),
        in_specs=[pl.BlockSpec((_RE, 2 * D), lambda i: (i, 0))],
        out_specs=pl.BlockSpec((_RE // 128, 128), lambda i: (i, 0)),
        out_shape=jax.ShapeDtypeStruct((ne // 128, 128), jnp.float32),
        compiler_params=pltpu.CompilerParams(
            dimension_semantics=("parallel",)),
    )(ab)
    return r2.reshape(ne)


_R = 2000


def _dis_block(deg_ref):
    deg = deg_ref[0, :, 0:1] + deg_ref[1, :, 0:1] + 1.0
    return lax.rsqrt(deg)


def _enc1_body(x_ref, w_ref, deg_ref, h0_ref, g1_ref):
    dis = _dis_block(deg_ref)
    h0 = jnp.dot(x_ref[...], w_ref[...], preferred_element_type=jnp.float32)
    h0_ref[...] = h0
    g1_ref[...] = h0 * dis


def _tc_encode1(x, W1, degp):
    return pl.pallas_call(
        _enc1_body,
        grid=(N // _R,),
        in_specs=[
            pl.BlockSpec((_R, 128), lambda i: (i, 0)),
            pl.BlockSpec((128, 128), lambda i: (0, 0)),
            pl.BlockSpec((NC, _R, 16), lambda i: (0, i, 0)),
        ],
        out_specs=[
            pl.BlockSpec((_R, 128), lambda i: (i, 0)),
            pl.BlockSpec((_R, 128), lambda i: (i, 0)),
        ],
        out_shape=[jax.ShapeDtypeStruct((N, 128), jnp.float32)] * 2,
    )(x, W1, degp)


def _enc23_body(s1_ref, h0_ref, deg_ref, b1_ref, eps_ref, w2_ref,
                h2_ref, g2_ref):
    dis = _dis_block(deg_ref)
    s = s1_ref[0] + s1_ref[1]
    h = jnp.maximum(dis * s + (dis * dis) * h0_ref[...] + b1_ref[...], 0.0)
    mean = jnp.sum(h, axis=0, keepdims=True) * (1.0 / N)
    var = jnp.sum(h * h, axis=0, keepdims=True) * (1.0 / N) - mean * mean
    std = jnp.sqrt(jnp.maximum(var, 0.0))
    hs = h + (mean + std * eps_ref[...])
    h2 = jnp.dot(hs, w2_ref[...], preferred_element_type=jnp.float32)
    h2_ref[...] = h2
    g2_ref[...] = h2 * dis


def _tc_encode23(s1p, h0, degp, b1, eps, W2):
    return pl.pallas_call(
        _enc23_body,
        grid=(1,),
        in_specs=[
            pl.BlockSpec((NC, N, 128), lambda i: (0, 0, 0)),
            pl.BlockSpec((N, 128), lambda i: (0, 0)),
            pl.BlockSpec((NC, N, 16), lambda i: (0, 0, 0)),
            pl.BlockSpec((1, 128), lambda i: (0, 0)),
            pl.BlockSpec((1, 128), lambda i: (0, 0)),
            pl.BlockSpec((128, 64), lambda i: (0, 0)),
        ],
        out_specs=[
            pl.BlockSpec((N, 64), lambda i: (0, 0)),
            pl.BlockSpec((N, 64), lambda i: (0, 0)),
        ],
        out_shape=[jax.ShapeDtypeStruct((N, 64), jnp.float32)] * 2,
    )(s1p, h0, degp, b1.reshape(1, 128), eps.reshape(1, 128), W2)


def _enc4_body(s2_ref, h2_ref, deg_ref, b2_ref, z_ref):
    dis = _dis_block(deg_ref)
    s = s2_ref[0] + s2_ref[1]
    z_ref[...] = dis * s + (dis * dis) * h2_ref[...] + b2_ref[...]


def _tc_encode4(s2p, h2, degp, b2):
    return pl.pallas_call(
        _enc4_body,
        grid=(N // _R,),
        in_specs=[
            pl.BlockSpec((NC, _R, 64), lambda i: (0, i, 0)),
            pl.BlockSpec((_R, 64), lambda i: (i, 0)),
            pl.BlockSpec((NC, _R, 16), lambda i: (0, i, 0)),
            pl.BlockSpec((1, 64), lambda i: (0, 0)),
        ],
        out_specs=pl.BlockSpec((_R, 64), lambda i: (i, 0)),
        out_shape=jax.ShapeDtypeStruct((N, 64), jnp.float32),
    )(s2p, h2, degp, b2.reshape(1, 64))


def kernel(x, edge_index, edge_label_index, W1, b1, W2, b2):
    src = jnp.asarray(edge_index[0], jnp.int32)
    dst = jnp.asarray(edge_index[1], jnp.int32)
    lsrc = jnp.asarray(edge_label_index[0], jnp.int32)
    ldst = jnp.asarray(edge_label_index[1], jnp.int32)
    eps = jax.random.normal(jax.random.key(42), (128,), dtype=jnp.float32)

    degp = _sc_degree(dst)
    h0, g1 = _tc_encode1(x, W1, degp)
    s1p = _sc_scatter_rows(g1, src, dst, 128)
    h2, g2 = _tc_encode23(s1p, h0, degp, b1, eps, W2)
    s2p = _sc_scatter_rows(g2, src, dst, 64)
    z = _tc_encode4(s2p, h2, degp, b2)
    ab1 = _sc_gather_pairs(z, lsrc, ldst, 64, 0, E // 2)
    ab2 = _sc_gather_pairs(z, lsrc, ldst, 64, E // 2, E // 2)
    r1 = _tc_dot(ab1, 64, E // 2)
    r2 = _tc_dot(ab2, 64, E // 2)
    return jnp.concatenate([r1, r2])
